# baseline (speedup 1.0000x reference)
import functools

import jax
import jax.numpy as jnp
from jax import lax
from jax.experimental import pallas as pl
from jax.experimental.pallas import tpu as pltpu
from jax.experimental.pallas import tpu_sc as plsc

N, E, R, D_IN, HEADS, D_OUT = 10000, 320000, 8, 128, 4, 32
K = HEADS * D_OUT
CHUNK = 128
NCHUNK = E // CHUNK
NW = 32
NPAD = 10240
TILE_ROWS = NPAD // 16
DEN_TILE = NPAD * HEADS // 16
BM = 400


def _mm_body(h_ref, w2_ref, wsd_ref, o1_ref, o2_ref):
    h = h_ref[...]
    o1_ref[...] = jnp.dot(
        h, w2_ref[...], preferred_element_type=jnp.float32
    ).astype(jnp.bfloat16)
    o2_ref[...] = jnp.dot(h, wsd_ref[...], preferred_element_type=jnp.float32)


def _mm_call(H, W2, Wsd):
    return pl.pallas_call(
        _mm_body,
        grid=(N // BM,),
        in_specs=[
            pl.BlockSpec((BM, D_IN), lambda i: (i, 0)),
            pl.BlockSpec((D_IN, R * K), lambda i: (0, 0)),
            pl.BlockSpec((D_IN, R * 8), lambda i: (0, 0)),
        ],
        out_specs=[
            pl.BlockSpec((BM, R * K), lambda i: (i, 0)),
            pl.BlockSpec((BM, R * 8), lambda i: (i, 0)),
        ],
        out_shape=[
            jax.ShapeDtypeStruct((N, R * K), jnp.bfloat16),
            jax.ShapeDtypeStruct((N, R * 8), jnp.float32),
        ],
    )(H, W2, Wsd)


_mesh = plsc.VectorSubcoreMesh(core_axis_name="c", subcore_axis_name="s")

NCH_W = (E // NW) // CHUNK
NTAIL = NCHUNK - NCH_W * NW


@functools.partial(
    pl.kernel,
    out_type=[
        jax.ShapeDtypeStruct((2, NPAD, K), jnp.float32),
        jax.ShapeDtypeStruct((2, NPAD * HEADS), jnp.float32),
    ],
    mesh=_mesh,
    compiler_params=pltpu.CompilerParams(use_tc_tiling_on_sc=False),
    scratch_types=[
        [pltpu.VMEM((CHUNK,), jnp.int32) for _ in range(2)],
        [pltpu.VMEM((CHUNK,), jnp.int32) for _ in range(2)],
        [pltpu.VMEM((CHUNK,), jnp.int32) for _ in range(2)],
        [pltpu.VMEM((CHUNK,), jnp.int32) for _ in range(2)],
        [pltpu.VMEM((CHUNK,), jnp.int32) for _ in range(2)],
        [[pltpu.VMEM((CHUNK,), jnp.int32) for _ in range(HEADS)]
         for _ in range(2)],
        [[pltpu.VMEM((CHUNK,), jnp.int32) for _ in range(HEADS)]
         for _ in range(2)],
        [[pltpu.VMEM((CHUNK,), jnp.int32) for _ in range(HEADS)]
         for _ in range(2)],
        [[pltpu.VMEM((CHUNK,), jnp.float32) for _ in range(HEADS)]
         for _ in range(2)],
        [[pltpu.VMEM((CHUNK,), jnp.float32) for _ in range(HEADS)]
         for _ in range(2)],
        [[pltpu.VMEM((CHUNK,), jnp.float32) for _ in range(HEADS)]
         for _ in range(2)],
        [pltpu.VMEM((CHUNK, K // 2), jnp.int32) for _ in range(2)],
        pltpu.VMEM((CHUNK, K), jnp.float32),
        pltpu.VMEM_SHARED((NPAD, K), jnp.float32),
        pltpu.VMEM_SHARED((NPAD * HEADS,), jnp.float32),
        [pltpu.SemaphoreType.DMA for _ in range(2)],
        [pltpu.SemaphoreType.DMA for _ in range(2)],
        [pltpu.SemaphoreType.DMA for _ in range(2)],
    ],
)
def _edge_kernel(src_hbm, dst_hbm, rt_hbm, hall_hbm, sdf_hbm, zacc_hbm,
                 zden_hbm, acc_out, den_out,
                 srcv, dstraw, rtv, fiv, dsc, sidx, didx, denidx,
                 svb, dvb, exb, hrow, msg, acc_sp, den_sp, isem, gsem, ssem):
    cid = lax.axis_index("c")
    sid = lax.axis_index("s")
    wid = sid * 2 + cid

    base = sid * TILE_ROWS
    dbase = sid * DEN_TILE
    pltpu.sync_copy(zacc_hbm, acc_sp.at[pl.ds(base, TILE_ROWS)])
    pltpu.sync_copy(zden_hbm, den_sp.at[pl.ds(dbase, DEN_TILE)])
    plsc.subcore_barrier()

    span0 = wid * (NCH_W * CHUNK)

    def idx_copies(t, b):
        e0 = span0 + t * CHUNK
        return [
            pltpu.make_async_copy(src_hbm.at[pl.ds(e0, CHUNK)], srcv[b],
                                  isem[b]),
            pltpu.make_async_copy(dst_hbm.at[pl.ds(e0, CHUNK)], dstraw[b],
                                  isem[b]),
            pltpu.make_async_copy(rt_hbm.at[pl.ds(e0, CHUNK)], rtv[b],
                                  isem[b]),
        ]

    def fire_idx(t, b):
        for c in idx_copies(t, b):
            c.start()

    def wait_idx(b):
        for c in idx_copies(0, b):
            c.wait()

    def fib(b):
        def g_body(g, c):
            sl = pl.ds(g * 16, 16)
            s16 = srcv[b][sl]
            d16 = dstraw[b][sl]
            r16 = rtv[b][sl]
            fi = s16 * R + r16
            fid = d16 * R + r16
            fiv[b][sl] = fi
            dsc[b][sl] = d16
            for h in range(HEADS):
                sidx[b][h][sl] = fi * 8 + h
                didx[b][h][sl] = fid * 8 + (4 + h)
                denidx[b][h][sl] = d16 * HEADS + h
            return c

        lax.fori_loop(0, CHUNK // 16, g_body, 0)

    def fire_gathers(b):
        pltpu.async_copy(hall_hbm.at[fiv[b]], hrow[b], gsem[b])
        for h in range(HEADS):
            pltpu.async_copy(sdf_hbm.at[sidx[b][h]], svb[b][h], gsem[b])
            pltpu.async_copy(sdf_hbm.at[didx[b][h]], dvb[b][h], gsem[b])

    def wait_gathers(b):
        pltpu.make_async_copy(hall_hbm.at[fiv[b]], hrow[b], gsem[b]).wait()
        for h in range(HEADS):
            pltpu.make_async_copy(sdf_hbm.at[sidx[b][h]], svb[b][h],
                                  gsem[b]).wait()
            pltpu.make_async_copy(sdf_hbm.at[didx[b][h]], dvb[b][h],
                                  gsem[b]).wait()

    def fire_scatters(b):
        pltpu.async_copy(msg, acc_sp.at[dsc[b]], ssem[b], add=True)
        for h in range(HEADS):
            pltpu.async_copy(exb[b][h], den_sp.at[denidx[b][h]], ssem[b],
                             add=True)

    def wait_scatters(b):
        pltpu.make_async_copy(msg, acc_sp.at[dsc[b]], ssem[b]).wait()
        for h in range(HEADS):
            pltpu.make_async_copy(exb[b][h], den_sp.at[denidx[b][h]],
                                  ssem[b]).wait()

    def exb_compute(b):
        def g_body(g, c):
            sl = pl.ds(g * 16, 16)
            for h in range(HEADS):
                logit = svb[b][h][sl] + dvb[b][h][sl]
                logit = jnp.maximum(logit, 0.2 * logit)
                exb[b][h][sl] = jnp.exp(logit)
            return c

        lax.fori_loop(0, CHUNK // 16, g_body, 0)

    def mb(b):
        def g_body(g, c):
            ws = [exb[b][h][pl.ds(g * 16, 16)] for h in range(HEADS)]

            def inner(o, c2):
                i = g * 16 + o
                sel = jnp.full((16,), o, jnp.int32)
                for h in range(HEADS):
                    eb = ws[h][sel]
                    u = hrow[b][i, pl.ds(h * 16, 16)]
                    a0 = lax.bitcast_convert_type(u << 16, jnp.float32)
                    a1 = lax.bitcast_convert_type(
                        u & jnp.int32(-65536), jnp.float32)
                    msg[i, pl.ds(h * 32, 16)] = a0 * eb
                    msg[i, pl.ds(h * 32 + 16, 16)] = a1 * eb
                return c2

            lax.fori_loop(0, 16, inner, c)
            return c

        lax.fori_loop(0, CHUNK // 16, g_body, 0)

    def step(t, b):
        wait_gathers(b)

        @pl.when(t >= 1)
        def _():
            wait_scatters(1 - b)

        @pl.when(t + 1 < NCH_W)
        def _():
            wait_idx(1 - b)
            fib(1 - b)
            fire_gathers(1 - b)

        @pl.when(t + 2 < NCH_W)
        def _():
            fire_idx(t + 2, b)

        exb_compute(b)
        mb(b)
        fire_scatters(b)

    fire_idx(0, 0)
    wait_idx(0)
    fib(0)
    fire_gathers(0)
    fire_idx(1, 1)

    def t2_body(t2, c):
        step(2 * t2, 0)
        step(2 * t2 + 1, 1)
        return c

    lax.fori_loop(0, NCH_W // 2, t2_body, 0)
    wait_scatters(1)

    @pl.when(wid < NTAIL)
    def _():
        e0 = NCH_W * NW * CHUNK + wid * CHUNK
        pltpu.sync_copy(src_hbm.at[pl.ds(e0, CHUNK)], srcv[0])
        pltpu.sync_copy(dst_hbm.at[pl.ds(e0, CHUNK)], dstraw[0])
        pltpu.sync_copy(rt_hbm.at[pl.ds(e0, CHUNK)], rtv[0])
        fib(0)
        fire_gathers(0)
        wait_gathers(0)
        exb_compute(0)
        mb(0)
        pltpu.sync_copy(msg, acc_sp.at[dsc[0]], add=True)
        for h in range(HEADS):
            pltpu.sync_copy(exb[0][h], den_sp.at[denidx[0][h]], add=True)

    plsc.subcore_barrier()
    pltpu.sync_copy(acc_sp.at[pl.ds(base, TILE_ROWS)],
                    acc_out.at[cid, pl.ds(base, TILE_ROWS)])
    pltpu.sync_copy(den_sp.at[pl.ds(dbase, DEN_TILE)],
                    den_out.at[cid, pl.ds(dbase, DEN_TILE)])


def _comb_body(acc_ref, den_ref, seg_ref, bias_ref, o_ref):
    a = acc_ref[0] + acc_ref[1]
    d = den_ref[0] + den_ref[1]
    db = jnp.dot(d, seg_ref[...], preferred_element_type=jnp.float32)
    o_ref[...] = a / (db + 1e-16) + bias_ref[...]


def _comb_call(acc, den, seg, bias2d):
    return pl.pallas_call(
        _comb_body,
        grid=(N // BM,),
        in_specs=[
            pl.BlockSpec((2, BM, K), lambda i: (0, i, 0)),
            pl.BlockSpec((2, BM, HEADS), lambda i: (0, i, 0)),
            pl.BlockSpec((HEADS, K), lambda i: (0, 0)),
            pl.BlockSpec((1, K), lambda i: (0, 0)),
        ],
        out_specs=pl.BlockSpec((BM, K), lambda i: (i, 0)),
        out_shape=jax.ShapeDtypeStruct((N, K), jnp.float32),
    )(acc, den, seg, bias2d)


def kernel(H, edge_index, edge_type, W, att_src, att_dst, bias):
    W2 = W.transpose(1, 0, 2).reshape(D_IN, R * K)
    p = jnp.arange(K)
    orig = 32 * (p // 32) + 16 * ((p % 32) % 2) + (p % 32) // 2
    perm = (jnp.arange(R)[:, None] * K + orig[None, :]).reshape(-1)
    W2 = W2[:, perm]
    Wr = W.reshape(R, D_IN, HEADS, D_OUT)
    ws = jnp.einsum('rdhj,rhj->rdh', Wr, att_src)
    wd = jnp.einsum('rdhj,rhj->rdh', Wr, att_dst)
    Wsd = jnp.concatenate([ws, wd], -1).transpose(1, 0, 2).reshape(D_IN, R * 8)

    hall2d, sd2d = _mm_call(H, W2, Wsd)
    hall = lax.bitcast_convert_type(
        hall2d.reshape(N * R, K // 2, 2), jnp.int32)
    sdf = sd2d.reshape(N * R * 8)

    src = edge_index[0]
    dst = edge_index[1]
    zacc = jnp.zeros((TILE_ROWS, K), jnp.float32)
    zden = jnp.zeros((DEN_TILE,), jnp.float32)
    acc, denf = _edge_kernel(src, dst, edge_type, hall, sdf, zacc, zden)
    den = denf.reshape(2, NPAD, HEADS)

    lanes = jnp.arange(K) // D_OUT
    seg = (lanes[None, :] == jnp.arange(HEADS)[:, None]).astype(jnp.float32)
    return _comb_call(acc, den, seg, bias.reshape(1, K))


# speedup vs baseline: 12.3890x; 12.3890x over previous
import functools

import jax
import jax.numpy as jnp
from jax import lax
from jax.experimental import pallas as pl
from jax.experimental.pallas import tpu as pltpu
from jax.experimental.pallas import tpu_sc as plsc

N, E, R, D_IN, HEADS, D_OUT = 10000, 320000, 8, 128, 4, 32
K = HEADS * D_OUT
CHUNK = 128
NCHUNK = E // CHUNK
NW = 32
NPAD = 10240
TILE_ROWS = NPAD // 16
DEN_TILE = NPAD * HEADS // 16
BM = 400


def _mm_body(h_ref, w2_ref, wsd_ref, o1_ref, o2_ref):
    h = h_ref[...]
    o1_ref[...] = jnp.dot(h, w2_ref[...], preferred_element_type=jnp.float32)
    o2_ref[...] = jnp.dot(h, wsd_ref[...], preferred_element_type=jnp.float32)


def _mm_call(H, W2, Wsd):
    return pl.pallas_call(
        _mm_body,
        grid=(N // BM,),
        in_specs=[
            pl.BlockSpec((BM, D_IN), lambda i: (i, 0)),
            pl.BlockSpec((D_IN, R * K), lambda i: (0, 0)),
            pl.BlockSpec((D_IN, R * 8), lambda i: (0, 0)),
        ],
        out_specs=[
            pl.BlockSpec((BM, R * K), lambda i: (i, 0)),
            pl.BlockSpec((BM, R * 8), lambda i: (i, 0)),
        ],
        out_shape=[
            jax.ShapeDtypeStruct((N, R * K), jnp.float32),
            jax.ShapeDtypeStruct((N, R * 8), jnp.float32),
        ],
    )(H, W2, Wsd)


_mesh = plsc.VectorSubcoreMesh(core_axis_name="c", subcore_axis_name="s")

NCH_W = (E // NW) // CHUNK
NTAIL = NCHUNK - NCH_W * NW


@functools.partial(
    pl.kernel,
    out_type=[
        jax.ShapeDtypeStruct((2, NPAD, K), jnp.float32),
        jax.ShapeDtypeStruct((2, NPAD * HEADS), jnp.float32),
    ],
    mesh=_mesh,
    scratch_types=[
        [pltpu.VMEM((CHUNK,), jnp.int32) for _ in range(2)],
        [pltpu.VMEM((CHUNK,), jnp.int32) for _ in range(2)],
        [pltpu.VMEM((CHUNK,), jnp.int32) for _ in range(2)],
        [pltpu.VMEM((CHUNK,), jnp.int32) for _ in range(2)],
        [pltpu.VMEM((CHUNK,), jnp.int32) for _ in range(2)],
        [[pltpu.VMEM((CHUNK,), jnp.int32) for _ in range(HEADS)]
         for _ in range(2)],
        [[pltpu.VMEM((CHUNK,), jnp.int32) for _ in range(HEADS)]
         for _ in range(2)],
        [[pltpu.VMEM((CHUNK,), jnp.int32) for _ in range(HEADS)]
         for _ in range(2)],
        [[pltpu.VMEM((CHUNK,), jnp.float32) for _ in range(HEADS)]
         for _ in range(2)],
        [[pltpu.VMEM((CHUNK,), jnp.float32) for _ in range(HEADS)]
         for _ in range(2)],
        [[pltpu.VMEM((CHUNK,), jnp.float32) for _ in range(HEADS)]
         for _ in range(2)],
        [pltpu.VMEM((CHUNK, K), jnp.float32) for _ in range(2)],
        pltpu.VMEM_SHARED((NPAD, K), jnp.float32),
        pltpu.VMEM_SHARED((NPAD * HEADS,), jnp.float32),
        [pltpu.SemaphoreType.DMA for _ in range(2)],
        [pltpu.SemaphoreType.DMA for _ in range(2)],
        [pltpu.SemaphoreType.DMA for _ in range(2)],
    ],
)
def _edge_kernel(src_hbm, dst_hbm, rt_hbm, hall_hbm, sdf_hbm, zacc_hbm,
                 zden_hbm, acc_out, den_out,
                 srcv, dstraw, rtv, fiv, dsc, sidx, didx, denidx,
                 svb, dvb, exb, hrow, acc_sp, den_sp, isem, gsem, ssem):
    cid = lax.axis_index("c")
    sid = lax.axis_index("s")
    wid = sid * 2 + cid

    base = sid * TILE_ROWS
    dbase = sid * DEN_TILE
    pltpu.sync_copy(zacc_hbm, acc_sp.at[pl.ds(base, TILE_ROWS)])
    pltpu.sync_copy(zden_hbm, den_sp.at[pl.ds(dbase, DEN_TILE)])
    plsc.subcore_barrier()

    span0 = wid * (NCH_W * CHUNK)

    def idx_copies(t, b):
        e0 = span0 + t * CHUNK
        return [
            pltpu.make_async_copy(src_hbm.at[pl.ds(e0, CHUNK)], srcv[b],
                                  isem[b]),
            pltpu.make_async_copy(dst_hbm.at[pl.ds(e0, CHUNK)], dstraw[b],
                                  isem[b]),
            pltpu.make_async_copy(rt_hbm.at[pl.ds(e0, CHUNK)], rtv[b],
                                  isem[b]),
        ]

    def fire_idx(t, b):
        for c in idx_copies(t, b):
            c.start()

    def wait_idx(b):
        for c in idx_copies(0, b):
            c.wait()

    def fib(b):
        def g_body(g, c):
            sl = pl.ds(g * 16, 16)
            s16 = srcv[b][sl]
            d16 = dstraw[b][sl]
            r16 = rtv[b][sl]
            fi = s16 * R + r16
            fid = d16 * R + r16
            fiv[b][sl] = fi
            dsc[b][sl] = d16
            for h in range(HEADS):
                sidx[b][h][sl] = fi * 8 + h
                didx[b][h][sl] = fid * 8 + (4 + h)
                denidx[b][h][sl] = d16 * HEADS + h
            return c

        lax.fori_loop(0, CHUNK // 16, g_body, 0)

    def fire_gathers(b):
        pltpu.async_copy(hall_hbm.at[fiv[b]], hrow[b], gsem[b])
        for h in range(HEADS):
            pltpu.async_copy(sdf_hbm.at[sidx[b][h]], svb[b][h], gsem[b])
            pltpu.async_copy(sdf_hbm.at[didx[b][h]], dvb[b][h], gsem[b])

    def wait_gathers(b):
        pltpu.make_async_copy(hall_hbm.at[fiv[b]], hrow[b], gsem[b]).wait()
        for h in range(HEADS):
            pltpu.make_async_copy(sdf_hbm.at[sidx[b][h]], svb[b][h],
                                  gsem[b]).wait()
            pltpu.make_async_copy(sdf_hbm.at[didx[b][h]], dvb[b][h],
                                  gsem[b]).wait()

    def fire_scatters(b):
        pltpu.async_copy(hrow[b], acc_sp.at[dsc[b]], ssem[b], add=True)
        for h in range(HEADS):
            pltpu.async_copy(exb[b][h], den_sp.at[denidx[b][h]], ssem[b],
                             add=True)

    def wait_scatters(b):
        pltpu.make_async_copy(hrow[b], acc_sp.at[dsc[b]], ssem[b]).wait()
        for h in range(HEADS):
            pltpu.make_async_copy(exb[b][h], den_sp.at[denidx[b][h]],
                                  ssem[b]).wait()

    def exb_compute(b):
        def g_body(g, c):
            sl = pl.ds(g * 16, 16)
            for h in range(HEADS):
                logit = svb[b][h][sl] + dvb[b][h][sl]
                logit = jnp.maximum(logit, 0.2 * logit)
                exb[b][h][sl] = jnp.exp(logit)
            return c

        lax.fori_loop(0, CHUNK // 16, g_body, 0)

    def mb(b):
        def g_body(g, c):
            ws = [exb[b][h][pl.ds(g * 16, 16)] for h in range(HEADS)]

            def inner(o, c2):
                i = g * 16 + o
                sel = jnp.full((16,), o, jnp.int32)
                for h in range(HEADS):
                    eb = ws[h][sel]
                    for cc in range(2):
                        col = h * 32 + cc * 16
                        hrow[b][i, pl.ds(col, 16)] = (
                            hrow[b][i, pl.ds(col, 16)] * eb)
                return c2

            lax.fori_loop(0, 16, inner, c)
            return c

        lax.fori_loop(0, CHUNK // 16, g_body, 0)

    def step(t, b):
        wait_gathers(b)

        @pl.when(t >= 1)
        def _():
            wait_scatters(1 - b)

        @pl.when(t + 1 < NCH_W)
        def _():
            wait_idx(1 - b)
            fib(1 - b)
            fire_gathers(1 - b)

        @pl.when(t + 2 < NCH_W)
        def _():
            fire_idx(t + 2, b)

        exb_compute(b)
        mb(b)
        fire_scatters(b)

    fire_idx(0, 0)
    wait_idx(0)
    fib(0)
    fire_gathers(0)
    fire_idx(1, 1)

    def t2_body(t2, c):
        step(2 * t2, 0)
        step(2 * t2 + 1, 1)
        return c

    lax.fori_loop(0, NCH_W // 2, t2_body, 0)
    wait_scatters(1)

    @pl.when(wid < NTAIL)
    def _():
        e0 = NCH_W * NW * CHUNK + wid * CHUNK
        pltpu.sync_copy(src_hbm.at[pl.ds(e0, CHUNK)], srcv[0])
        pltpu.sync_copy(dst_hbm.at[pl.ds(e0, CHUNK)], dstraw[0])
        pltpu.sync_copy(rt_hbm.at[pl.ds(e0, CHUNK)], rtv[0])
        fib(0)
        fire_gathers(0)
        wait_gathers(0)
        exb_compute(0)
        mb(0)
        pltpu.sync_copy(hrow[0], acc_sp.at[dsc[0]], add=True)
        for h in range(HEADS):
            pltpu.sync_copy(exb[0][h], den_sp.at[denidx[0][h]], add=True)

    plsc.subcore_barrier()
    pltpu.sync_copy(acc_sp.at[pl.ds(base, TILE_ROWS)],
                    acc_out.at[cid, pl.ds(base, TILE_ROWS)])
    pltpu.sync_copy(den_sp.at[pl.ds(dbase, DEN_TILE)],
                    den_out.at[cid, pl.ds(dbase, DEN_TILE)])


def _comb_body(acc_ref, den_ref, seg_ref, bias_ref, o_ref):
    a = acc_ref[0] + acc_ref[1]
    d = den_ref[0] + den_ref[1]
    db = jnp.dot(d, seg_ref[...], preferred_element_type=jnp.float32)
    o_ref[...] = a / (db + 1e-16) + bias_ref[...]


def _comb_call(acc, den, seg, bias2d):
    return pl.pallas_call(
        _comb_body,
        grid=(N // BM,),
        in_specs=[
            pl.BlockSpec((2, BM, K), lambda i: (0, i, 0)),
            pl.BlockSpec((2, BM, HEADS), lambda i: (0, i, 0)),
            pl.BlockSpec((HEADS, K), lambda i: (0, 0)),
            pl.BlockSpec((1, K), lambda i: (0, 0)),
        ],
        out_specs=pl.BlockSpec((BM, K), lambda i: (i, 0)),
        out_shape=jax.ShapeDtypeStruct((N, K), jnp.float32),
    )(acc, den, seg, bias2d)


def kernel(H, edge_index, edge_type, W, att_src, att_dst, bias):
    W2 = W.transpose(1, 0, 2).reshape(D_IN, R * K)
    Wr = W.reshape(R, D_IN, HEADS, D_OUT)
    ws = jnp.einsum('rdhj,rhj->rdh', Wr, att_src)
    wd = jnp.einsum('rdhj,rhj->rdh', Wr, att_dst)
    Wsd = jnp.concatenate([ws, wd], -1).transpose(1, 0, 2).reshape(D_IN, R * 8)

    hall2d, sd2d = _mm_call(H, W2, Wsd)
    hall = hall2d.reshape(N * R, K)
    sdf = sd2d.reshape(N * R * 8)

    src = edge_index[0]
    dst = edge_index[1]
    zacc = jnp.zeros((TILE_ROWS, K), jnp.float32)
    zden = jnp.zeros((DEN_TILE,), jnp.float32)
    acc, denf = _edge_kernel(src, dst, edge_type, hall, sdf, zacc, zden)
    den = denf.reshape(2, NPAD, HEADS)

    lanes = jnp.arange(K) // D_OUT
    seg = (lanes[None, :] == jnp.arange(HEADS)[:, None]).astype(jnp.float32)
    return _comb_call(acc, den, seg, bias.reshape(1, K))


# speedup vs baseline: 13.0413x; 1.0527x over previous
import functools

import jax
import jax.numpy as jnp
from jax import lax
from jax.experimental import pallas as pl
from jax.experimental.pallas import tpu as pltpu
from jax.experimental.pallas import tpu_sc as plsc

N, E, R, D_IN, HEADS, D_OUT = 10000, 320000, 8, 128, 4, 32
K = HEADS * D_OUT
CHUNK = 128
NCHUNK = E // CHUNK
NW = 32
NPAD = 10240
TILE_ROWS = NPAD // 16
DEN_TILE = NPAD * HEADS // 16
BM = 400


def _mm_body(h_ref, w2_ref, wsd_ref, o1_ref, o2_ref):
    h = h_ref[...]
    o1_ref[...] = jnp.dot(h, w2_ref[...], preferred_element_type=jnp.float32)
    o2_ref[...] = jnp.dot(h, wsd_ref[...], preferred_element_type=jnp.float32)


def _mm_call(H, W2, Wsd):
    return pl.pallas_call(
        _mm_body,
        grid=(N // BM,),
        in_specs=[
            pl.BlockSpec((BM, D_IN), lambda i: (i, 0)),
            pl.BlockSpec((D_IN, R * K), lambda i: (0, 0)),
            pl.BlockSpec((D_IN, R * 8), lambda i: (0, 0)),
        ],
        out_specs=[
            pl.BlockSpec((BM, R * K), lambda i: (i, 0)),
            pl.BlockSpec((BM, R * 8), lambda i: (i, 0)),
        ],
        out_shape=[
            jax.ShapeDtypeStruct((N, R * K), jnp.float32),
            jax.ShapeDtypeStruct((N, R * 8), jnp.float32),
        ],
    )(H, W2, Wsd)


_mesh = plsc.VectorSubcoreMesh(core_axis_name="c", subcore_axis_name="s")

NCH_W = (E // NW) // CHUNK
NTAIL = NCHUNK - NCH_W * NW


@functools.partial(
    pl.kernel,
    out_type=[
        jax.ShapeDtypeStruct((2, NPAD, K), jnp.float32),
        jax.ShapeDtypeStruct((2, NPAD * HEADS), jnp.float32),
    ],
    mesh=_mesh,
    scratch_types=[
        [pltpu.VMEM((CHUNK,), jnp.int32) for _ in range(2)],
        [pltpu.VMEM((CHUNK,), jnp.int32) for _ in range(2)],
        [pltpu.VMEM((CHUNK,), jnp.int32) for _ in range(2)],
        [pltpu.VMEM((CHUNK,), jnp.int32) for _ in range(2)],
        [pltpu.VMEM((CHUNK,), jnp.int32) for _ in range(2)],
        [[pltpu.VMEM((CHUNK,), jnp.int32) for _ in range(HEADS)]
         for _ in range(2)],
        [[pltpu.VMEM((CHUNK,), jnp.int32) for _ in range(HEADS)]
         for _ in range(2)],
        [[pltpu.VMEM((CHUNK,), jnp.int32) for _ in range(HEADS)]
         for _ in range(2)],
        [[pltpu.VMEM((CHUNK,), jnp.float32) for _ in range(HEADS)]
         for _ in range(2)],
        [[pltpu.VMEM((CHUNK,), jnp.float32) for _ in range(HEADS)]
         for _ in range(2)],
        [[pltpu.VMEM((CHUNK,), jnp.float32) for _ in range(HEADS)]
         for _ in range(2)],
        [pltpu.VMEM((CHUNK, K), jnp.float32) for _ in range(2)],
        pltpu.VMEM_SHARED((NPAD, K), jnp.float32),
        pltpu.VMEM_SHARED((NPAD * HEADS,), jnp.float32),
        [pltpu.SemaphoreType.DMA for _ in range(2)],
        [pltpu.SemaphoreType.DMA for _ in range(2)],
        [pltpu.SemaphoreType.DMA for _ in range(2)],
    ],
)
def _edge_kernel(ei_hbm, rt_hbm, hall_hbm, sdf_hbm,
                 zden_hbm, acc_out, den_out,
                 srcv, dstraw, rtv, fiv, dsc, sidx, didx, denidx,
                 svb, dvb, exb, hrow, acc_sp, den_sp, isem, gsem, ssem):
    cid = lax.axis_index("c")
    sid = lax.axis_index("s")
    wid = sid * 2 + cid

    base = sid * TILE_ROWS
    dbase = sid * DEN_TILE

    def zb(g, c):
        for cc in range(K // 16):
            hrow[0][g, pl.ds(cc * 16, 16)] = jnp.zeros((16,), jnp.float32)
        return c

    lax.fori_loop(0, CHUNK, zb, 0)
    for kk in range(TILE_ROWS // CHUNK):
        pltpu.sync_copy(hrow[0], acc_sp.at[pl.ds(base + kk * CHUNK, CHUNK)])
    pltpu.sync_copy(zden_hbm, den_sp.at[pl.ds(dbase, DEN_TILE)])
    plsc.subcore_barrier()

    span0 = wid * (NCH_W * CHUNK)

    def idx_copies(t, b):
        e0 = span0 + t * CHUNK
        return [
            pltpu.make_async_copy(ei_hbm.at[0, pl.ds(e0, CHUNK)], srcv[b],
                                  isem[b]),
            pltpu.make_async_copy(ei_hbm.at[1, pl.ds(e0, CHUNK)], dstraw[b],
                                  isem[b]),
            pltpu.make_async_copy(rt_hbm.at[pl.ds(e0, CHUNK)], rtv[b],
                                  isem[b]),
        ]

    def fire_idx(t, b):
        for c in idx_copies(t, b):
            c.start()

    def wait_idx(b):
        for c in idx_copies(0, b):
            c.wait()

    def fib(b):
        def g_body(g, c):
            sl = pl.ds(g * 16, 16)
            s16 = srcv[b][sl]
            d16 = dstraw[b][sl]
            r16 = rtv[b][sl]
            fi = s16 * R + r16
            fid = d16 * R + r16
            fiv[b][sl] = fi
            dsc[b][sl] = d16
            for h in range(HEADS):
                sidx[b][h][sl] = fi * 8 + h
                didx[b][h][sl] = fid * 8 + (4 + h)
                denidx[b][h][sl] = d16 * HEADS + h
            return c

        lax.fori_loop(0, CHUNK // 16, g_body, 0)

    def fire_gathers(b):
        pltpu.async_copy(hall_hbm.at[fiv[b]], hrow[b], gsem[b])
        for h in range(HEADS):
            pltpu.async_copy(sdf_hbm.at[sidx[b][h]], svb[b][h], gsem[b])
            pltpu.async_copy(sdf_hbm.at[didx[b][h]], dvb[b][h], gsem[b])

    def wait_gathers(b):
        pltpu.make_async_copy(hall_hbm.at[fiv[b]], hrow[b], gsem[b]).wait()
        for h in range(HEADS):
            pltpu.make_async_copy(sdf_hbm.at[sidx[b][h]], svb[b][h],
                                  gsem[b]).wait()
            pltpu.make_async_copy(sdf_hbm.at[didx[b][h]], dvb[b][h],
                                  gsem[b]).wait()

    def fire_scatters(b):
        pltpu.async_copy(hrow[b], acc_sp.at[dsc[b]], ssem[b], add=True)
        for h in range(HEADS):
            pltpu.async_copy(exb[b][h], den_sp.at[denidx[b][h]], ssem[b],
                             add=True)

    def wait_scatters(b):
        pltpu.make_async_copy(hrow[b], acc_sp.at[dsc[b]], ssem[b]).wait()
        for h in range(HEADS):
            pltpu.make_async_copy(exb[b][h], den_sp.at[denidx[b][h]],
                                  ssem[b]).wait()

    def exb_compute(b):
        def g_body(g, c):
            sl = pl.ds(g * 16, 16)
            for h in range(HEADS):
                logit = svb[b][h][sl] + dvb[b][h][sl]
                logit = jnp.maximum(logit, 0.2 * logit)
                exb[b][h][sl] = jnp.exp(logit)
            return c

        lax.fori_loop(0, CHUNK // 16, g_body, 0)

    def mb(b):
        def g_body(g, c):
            ws = [exb[b][h][pl.ds(g * 16, 16)] for h in range(HEADS)]

            def inner(o, c2):
                i = g * 16 + o
                sel = jnp.full((16,), o, jnp.int32)
                for h in range(HEADS):
                    eb = ws[h][sel]
                    for cc in range(2):
                        col = h * 32 + cc * 16
                        hrow[b][i, pl.ds(col, 16)] = (
                            hrow[b][i, pl.ds(col, 16)] * eb)
                return c2

            lax.fori_loop(0, 16, inner, c)
            return c

        lax.fori_loop(0, CHUNK // 16, g_body, 0)

    def step(t, b):
        wait_gathers(b)

        @pl.when(t >= 1)
        def _():
            wait_scatters(1 - b)

        @pl.when(t + 1 < NCH_W)
        def _():
            wait_idx(1 - b)
            fib(1 - b)
            fire_gathers(1 - b)

        @pl.when(t + 2 < NCH_W)
        def _():
            fire_idx(t + 2, b)

        exb_compute(b)
        mb(b)
        fire_scatters(b)

    fire_idx(0, 0)
    wait_idx(0)
    fib(0)
    fire_gathers(0)
    fire_idx(1, 1)

    def t2_body(t2, c):
        step(2 * t2, 0)
        step(2 * t2 + 1, 1)
        return c

    lax.fori_loop(0, NCH_W // 2, t2_body, 0)
    wait_scatters(1)

    @pl.when(wid < NTAIL)
    def _():
        e0 = NCH_W * NW * CHUNK + wid * CHUNK
        pltpu.sync_copy(ei_hbm.at[0, pl.ds(e0, CHUNK)], srcv[0])
        pltpu.sync_copy(ei_hbm.at[1, pl.ds(e0, CHUNK)], dstraw[0])
        pltpu.sync_copy(rt_hbm.at[pl.ds(e0, CHUNK)], rtv[0])
        fib(0)
        fire_gathers(0)
        wait_gathers(0)
        exb_compute(0)
        mb(0)
        pltpu.sync_copy(hrow[0], acc_sp.at[dsc[0]], add=True)
        for h in range(HEADS):
            pltpu.sync_copy(exb[0][h], den_sp.at[denidx[0][h]], add=True)

    plsc.subcore_barrier()
    pltpu.sync_copy(acc_sp.at[pl.ds(base, TILE_ROWS)],
                    acc_out.at[cid, pl.ds(base, TILE_ROWS)])
    pltpu.sync_copy(den_sp.at[pl.ds(dbase, DEN_TILE)],
                    den_out.at[cid, pl.ds(dbase, DEN_TILE)])


def _comb_body(acc_ref, den_ref, seg_ref, bias_ref, o_ref):
    a = acc_ref[0] + acc_ref[1]
    d = den_ref[0] + den_ref[1]
    db = jnp.dot(d, seg_ref[...], preferred_element_type=jnp.float32)
    o_ref[...] = a / (db + 1e-16) + bias_ref[...]


def _comb_call(acc, den, seg, bias2d):
    return pl.pallas_call(
        _comb_body,
        grid=(N // BM,),
        in_specs=[
            pl.BlockSpec((2, BM, K), lambda i: (0, i, 0)),
            pl.BlockSpec((2, BM, HEADS), lambda i: (0, i, 0)),
            pl.BlockSpec((HEADS, K), lambda i: (0, 0)),
            pl.BlockSpec((1, K), lambda i: (0, 0)),
        ],
        out_specs=pl.BlockSpec((BM, K), lambda i: (i, 0)),
        out_shape=jax.ShapeDtypeStruct((N, K), jnp.float32),
    )(acc, den, seg, bias2d)


def kernel(H, edge_index, edge_type, W, att_src, att_dst, bias):
    W2 = W.transpose(1, 0, 2).reshape(D_IN, R * K)
    Wr = W.reshape(R, D_IN, HEADS, D_OUT)
    ws = jnp.einsum('rdhj,rhj->rdh', Wr, att_src)
    wd = jnp.einsum('rdhj,rhj->rdh', Wr, att_dst)
    Wsd = jnp.concatenate([ws, wd], -1).transpose(1, 0, 2).reshape(D_IN, R * 8)

    hall2d, sd2d = _mm_call(H, W2, Wsd)
    hall = hall2d.reshape(N * R, K)
    sdf = sd2d.reshape(N * R * 8)

    zden = jnp.zeros((DEN_TILE,), jnp.float32)
    acc, denf = _edge_kernel(edge_index, edge_type, hall, sdf, zden)
    den = denf.reshape(2, NPAD, HEADS)

    lanes = jnp.arange(K) // D_OUT
    seg = (lanes[None, :] == jnp.arange(HEADS)[:, None]).astype(jnp.float32)
    return _comb_call(acc, den, seg, bias.reshape(1, K))


# speedup vs baseline: 14.0629x; 1.0783x over previous
import functools

import jax
import jax.numpy as jnp
from jax import lax
from jax.experimental import pallas as pl
from jax.experimental.pallas import tpu as pltpu
from jax.experimental.pallas import tpu_sc as plsc

N, E, R, D_IN, HEADS, D_OUT = 10000, 320000, 8, 128, 4, 32
K = HEADS * D_OUT
CHUNK = 128
NCHUNK = E // CHUNK
NW = 32
NPAD = 10240
TILE_ROWS = NPAD // 16
DEN_TILE = NPAD * HEADS // 16
BM = 400


def _mm_body(h_ref, w2_ref, wsd_ref, o1_ref, o2_ref):
    h = h_ref[...]
    o1_ref[...] = jnp.dot(h, w2_ref[...], preferred_element_type=jnp.float32)
    o2_ref[...] = jnp.dot(
        h, wsd_ref[...], preferred_element_type=jnp.float32
    ).astype(jnp.bfloat16)


def _mm_call(H, W2, Wsd):
    return pl.pallas_call(
        _mm_body,
        grid=(N // BM,),
        in_specs=[
            pl.BlockSpec((BM, D_IN), lambda i: (i, 0)),
            pl.BlockSpec((D_IN, R * K), lambda i: (0, 0)),
            pl.BlockSpec((D_IN, R * 8), lambda i: (0, 0)),
        ],
        out_specs=[
            pl.BlockSpec((BM, R * K), lambda i: (i, 0)),
            pl.BlockSpec((BM, R * 8), lambda i: (i, 0)),
        ],
        out_shape=[
            jax.ShapeDtypeStruct((N, R * K), jnp.float32),
            jax.ShapeDtypeStruct((N, R * 8), jnp.bfloat16),
        ],
    )(H, W2, Wsd)


_mesh = plsc.VectorSubcoreMesh(core_axis_name="c", subcore_axis_name="s")

NCH_W = (E // NW) // CHUNK
NTAIL = NCHUNK - NCH_W * NW


@functools.partial(
    pl.kernel,
    out_type=[
        jax.ShapeDtypeStruct((2, NPAD, K), jnp.float32),
        jax.ShapeDtypeStruct((2, NPAD * HEADS), jnp.float32),
    ],
    mesh=_mesh,
    scratch_types=[
        [pltpu.VMEM((CHUNK,), jnp.int32) for _ in range(2)],
        [pltpu.VMEM((CHUNK,), jnp.int32) for _ in range(2)],
        [pltpu.VMEM((CHUNK,), jnp.int32) for _ in range(2)],
        [pltpu.VMEM((CHUNK,), jnp.int32) for _ in range(2)],
        [pltpu.VMEM((CHUNK,), jnp.int32) for _ in range(2)],
        [[pltpu.VMEM((CHUNK,), jnp.int32) for _ in range(2)]
         for _ in range(2)],
        [[pltpu.VMEM((CHUNK,), jnp.int32) for _ in range(2)]
         for _ in range(2)],
        [[pltpu.VMEM((CHUNK,), jnp.int32) for _ in range(HEADS)]
         for _ in range(2)],
        [[pltpu.VMEM((CHUNK,), jnp.int32) for _ in range(2)]
         for _ in range(2)],
        [[pltpu.VMEM((CHUNK,), jnp.int32) for _ in range(2)]
         for _ in range(2)],
        [[pltpu.VMEM((CHUNK,), jnp.float32) for _ in range(HEADS)]
         for _ in range(2)],
        [pltpu.VMEM((CHUNK, K), jnp.float32) for _ in range(2)],
        pltpu.VMEM_SHARED((NPAD, K), jnp.float32),
        pltpu.VMEM_SHARED((NPAD * HEADS,), jnp.float32),
        [pltpu.SemaphoreType.DMA for _ in range(2)],
        [pltpu.SemaphoreType.DMA for _ in range(2)],
        [pltpu.SemaphoreType.DMA for _ in range(2)],
    ],
)
def _edge_kernel(ei_hbm, rt_hbm, hall_hbm, sdf_hbm,
                 zden_hbm, acc_out, den_out,
                 srcv, dstraw, rtv, fiv, dsc, sidx, didx, denidx,
                 svb, dvb, exb, hrow, acc_sp, den_sp, isem, gsem, ssem):
    cid = lax.axis_index("c")
    sid = lax.axis_index("s")
    wid = sid * 2 + cid

    base = sid * TILE_ROWS
    dbase = sid * DEN_TILE

    def zb(g, c):
        for cc in range(K // 16):
            hrow[0][g, pl.ds(cc * 16, 16)] = jnp.zeros((16,), jnp.float32)
        return c

    lax.fori_loop(0, CHUNK, zb, 0)
    for kk in range(TILE_ROWS // CHUNK):
        pltpu.sync_copy(hrow[0], acc_sp.at[pl.ds(base + kk * CHUNK, CHUNK)])
    pltpu.sync_copy(zden_hbm, den_sp.at[pl.ds(dbase, DEN_TILE)])
    plsc.subcore_barrier()

    span0 = wid * (NCH_W * CHUNK)

    def idx_copies(t, b):
        e0 = span0 + t * CHUNK
        return [
            pltpu.make_async_copy(ei_hbm.at[0, pl.ds(e0, CHUNK)], srcv[b],
                                  isem[b]),
            pltpu.make_async_copy(ei_hbm.at[1, pl.ds(e0, CHUNK)], dstraw[b],
                                  isem[b]),
            pltpu.make_async_copy(rt_hbm.at[pl.ds(e0, CHUNK)], rtv[b],
                                  isem[b]),
        ]

    def fire_idx(t, b):
        for c in idx_copies(t, b):
            c.start()

    def wait_idx(b):
        for c in idx_copies(0, b):
            c.wait()

    def fib(b):
        def g_body(g, c):
            sl = pl.ds(g * 16, 16)
            s16 = srcv[b][sl]
            d16 = dstraw[b][sl]
            r16 = rtv[b][sl]
            fi = s16 * R + r16
            fid = d16 * R + r16
            fiv[b][sl] = fi
            dsc[b][sl] = d16
            for q in range(2):
                sidx[b][q][sl] = fi * 4 + q
                didx[b][q][sl] = fid * 4 + (2 + q)
            for h in range(HEADS):
                denidx[b][h][sl] = d16 * HEADS + h
            return c

        lax.fori_loop(0, CHUNK // 16, g_body, 0)

    def fire_gathers(b):
        pltpu.async_copy(hall_hbm.at[fiv[b]], hrow[b], gsem[b])
        for q in range(2):
            pltpu.async_copy(sdf_hbm.at[sidx[b][q]], svb[b][q], gsem[b])
            pltpu.async_copy(sdf_hbm.at[didx[b][q]], dvb[b][q], gsem[b])

    def wait_gathers(b):
        pltpu.make_async_copy(hall_hbm.at[fiv[b]], hrow[b], gsem[b]).wait()
        for q in range(2):
            pltpu.make_async_copy(sdf_hbm.at[sidx[b][q]], svb[b][q],
                                  gsem[b]).wait()
            pltpu.make_async_copy(sdf_hbm.at[didx[b][q]], dvb[b][q],
                                  gsem[b]).wait()

    def fire_scatters(b):
        pltpu.async_copy(hrow[b], acc_sp.at[dsc[b]], ssem[b], add=True)
        for h in range(HEADS):
            pltpu.async_copy(exb[b][h], den_sp.at[denidx[b][h]], ssem[b],
                             add=True)

    def wait_scatters(b):
        pltpu.make_async_copy(hrow[b], acc_sp.at[dsc[b]], ssem[b]).wait()
        for h in range(HEADS):
            pltpu.make_async_copy(exb[b][h], den_sp.at[denidx[b][h]],
                                  ssem[b]).wait()

    def exb_compute(b):
        hi = jnp.int32(-65536)

        def g_body(g, c):
            sl = pl.ds(g * 16, 16)
            sv, dv = [], []
            for q in range(2):
                u = svb[b][q][sl]
                v = dvb[b][q][sl]
                sv.append(lax.bitcast_convert_type(u << 16, jnp.float32))
                sv.append(lax.bitcast_convert_type(u & hi, jnp.float32))
                dv.append(lax.bitcast_convert_type(v << 16, jnp.float32))
                dv.append(lax.bitcast_convert_type(v & hi, jnp.float32))
            for h in range(HEADS):
                logit = sv[h] + dv[h]
                logit = jnp.maximum(logit, 0.2 * logit)
                exb[b][h][sl] = jnp.exp(logit)
            return c

        lax.fori_loop(0, CHUNK // 16, g_body, 0)

    def mb(b):
        def g_body(g, c):
            ws = [exb[b][h][pl.ds(g * 16, 16)] for h in range(HEADS)]

            def inner(o, c2):
                i = g * 16 + o
                sel = jnp.full((16,), o, jnp.int32)
                for h in range(HEADS):
                    eb = ws[h][sel]
                    for cc in range(2):
                        col = h * 32 + cc * 16
                        hrow[b][i, pl.ds(col, 16)] = (
                            hrow[b][i, pl.ds(col, 16)] * eb)
                return c2

            lax.fori_loop(0, 16, inner, c)
            return c

        lax.fori_loop(0, CHUNK // 16, g_body, 0)

    def step(t, b):
        wait_gathers(b)

        @pl.when(t >= 1)
        def _():
            wait_scatters(1 - b)

        @pl.when(t + 1 < NCH_W)
        def _():
            wait_idx(1 - b)
            fib(1 - b)
            fire_gathers(1 - b)

        @pl.when(t + 2 < NCH_W)
        def _():
            fire_idx(t + 2, b)

        exb_compute(b)
        mb(b)
        fire_scatters(b)

    fire_idx(0, 0)
    wait_idx(0)
    fib(0)
    fire_gathers(0)
    fire_idx(1, 1)

    def t2_body(t2, c):
        step(2 * t2, 0)
        step(2 * t2 + 1, 1)
        return c

    lax.fori_loop(0, NCH_W // 2, t2_body, 0)
    wait_scatters(1)

    @pl.when(wid < NTAIL)
    def _():
        e0 = NCH_W * NW * CHUNK + wid * CHUNK
        pltpu.sync_copy(ei_hbm.at[0, pl.ds(e0, CHUNK)], srcv[0])
        pltpu.sync_copy(ei_hbm.at[1, pl.ds(e0, CHUNK)], dstraw[0])
        pltpu.sync_copy(rt_hbm.at[pl.ds(e0, CHUNK)], rtv[0])
        fib(0)
        fire_gathers(0)
        wait_gathers(0)
        exb_compute(0)
        mb(0)
        pltpu.sync_copy(hrow[0], acc_sp.at[dsc[0]], add=True)
        for h in range(HEADS):
            pltpu.sync_copy(exb[0][h], den_sp.at[denidx[0][h]], add=True)

    plsc.subcore_barrier()
    pltpu.sync_copy(acc_sp.at[pl.ds(base, TILE_ROWS)],
                    acc_out.at[cid, pl.ds(base, TILE_ROWS)])
    pltpu.sync_copy(den_sp.at[pl.ds(dbase, DEN_TILE)],
                    den_out.at[cid, pl.ds(dbase, DEN_TILE)])


def _comb_body(acc_ref, den_ref, seg_ref, bias_ref, o_ref):
    a = acc_ref[0] + acc_ref[1]
    d = den_ref[0] + den_ref[1]
    db = jnp.dot(d, seg_ref[...], preferred_element_type=jnp.float32)
    o_ref[...] = a / (db + 1e-16) + bias_ref[...]


def _comb_call(acc, den, seg, bias2d):
    return pl.pallas_call(
        _comb_body,
        grid=(N // BM,),
        in_specs=[
            pl.BlockSpec((2, BM, K), lambda i: (0, i, 0)),
            pl.BlockSpec((2, BM, HEADS), lambda i: (0, i, 0)),
            pl.BlockSpec((HEADS, K), lambda i: (0, 0)),
            pl.BlockSpec((1, K), lambda i: (0, 0)),
        ],
        out_specs=pl.BlockSpec((BM, K), lambda i: (i, 0)),
        out_shape=jax.ShapeDtypeStruct((N, K), jnp.float32),
    )(acc, den, seg, bias2d)


def kernel(H, edge_index, edge_type, W, att_src, att_dst, bias):
    W2 = W.transpose(1, 0, 2).reshape(D_IN, R * K)
    Wr = W.reshape(R, D_IN, HEADS, D_OUT)
    ws = jnp.einsum('rdhj,rhj->rdh', Wr, att_src)
    wd = jnp.einsum('rdhj,rhj->rdh', Wr, att_dst)
    Wsd = jnp.concatenate([ws, wd], -1).transpose(1, 0, 2).reshape(D_IN, R * 8)

    hall2d, sd2d = _mm_call(H, W2, Wsd)
    hall = hall2d.reshape(N * R, K)
    sdf = lax.bitcast_convert_type(
        sd2d.reshape(N, R * 4, 2), jnp.int32
    ).reshape(N * R * 4)

    zden = jnp.zeros((DEN_TILE,), jnp.float32)
    acc, denf = _edge_kernel(edge_index, edge_type, hall, sdf, zden)
    den = denf.reshape(2, NPAD, HEADS)

    lanes = jnp.arange(K) // D_OUT
    seg = (lanes[None, :] == jnp.arange(HEADS)[:, None]).astype(jnp.float32)
    return _comb_call(acc, den, seg, bias.reshape(1, K))
